# baseline (speedup 1.0000x reference)
import functools

import jax
import jax.numpy as jnp
from jax import lax
from jax.experimental import pallas as pl
from jax.experimental.pallas import tpu as pltpu
from jax.experimental.pallas import tpu_sc as plsc

N = 10000
E = 320000
D = 128
HALF = E // 2

NC = 2
NS = 16
LANES = 16

CHUNK = 128
NCHUNKS = HALF // CHUNK
ZROWS = 80
NZBLK = N // ZROWS
WBLK = 624
WTAIL = N - NS * WBLK


def _sc_mesh():
    return plsc.VectorSubcoreMesh(core_axis_name="c", subcore_axis_name="s")


NCHUNKS_TOT = E // CHUNK
NBUF = 2
ZBLK = N // CHUNK
ZTAIL = N - ZBLK * CHUNK


@functools.partial(
    pl.kernel,
    out_type=(
        jax.ShapeDtypeStruct((N, D), jnp.float32),
        jax.ShapeDtypeStruct((N, D), jnp.float32),
    ),
    mesh=_sc_mesh(),
    scratch_types=(
        [pltpu.VMEM((3, CHUNK), jnp.int32)] * NBUF
        + [pltpu.VMEM((2, CHUNK), jnp.float32)] * NBUF
        + [pltpu.VMEM((CHUNK, D), jnp.float32)] * NBUF
        + [pltpu.VMEM((CHUNK,), jnp.int32)] * NBUF
        + [
            pltpu.VMEM((41, D), jnp.float32),
            pltpu.VMEM_SHARED((N, D), jnp.float32),
        ]
        + [pltpu.SemaphoreType.DMA] * (3 * NBUF)
    ),
)
def _sc_aggregate(x_hbm, rall_hbm, eidx_hbm, escale_hbm,
                  out_in, out_out, *scratch):
    idx_v = scratch[0:NBUF]
    scl_v = scratch[NBUF:2 * NBUF]
    rows_v = scratch[2 * NBUF:3 * NBUF]
    dst_v = scratch[3 * NBUF:4 * NBUF]
    rtab_v, acc_sh = scratch[4 * NBUF:4 * NBUF + 2]
    semM = scratch[4 * NBUF + 2:4 * NBUF + 2 + NBUF]
    semG = scratch[4 * NBUF + 2 + NBUF:4 * NBUF + 2 + 2 * NBUF]
    semS = scratch[4 * NBUF + 2 + 2 * NBUF:4 * NBUF + 2 + 3 * NBUF]

    cid = lax.axis_index("c")
    sid = lax.axis_index("s")

    nloc = (NCHUNKS - sid + NS - 1) // NS

    def chunk_idx(i):
        return cid * NCHUNKS + sid + i * NS

    def fire_meta(i, b):
        c = chunk_idx(i)
        pltpu.async_copy(eidx_hbm.at[c], idx_v[b], semM[b])
        pltpu.async_copy(escale_hbm.at[c], scl_v[b], semM[b])

    def wait_meta(i, b):
        c = chunk_idx(i)
        pltpu.make_async_copy(eidx_hbm.at[c], idx_v[b], semM[b]).wait()
        pltpu.make_async_copy(escale_hbm.at[c], scl_v[b], semM[b]).wait()

    for b in range(NBUF):
        fire_meta(b, b)

    zvec = jnp.zeros((LANES,), jnp.float32)

    def zrow(i, _):
        for d in range(D // LANES):
            rows_v[1][i, pl.ds(d * LANES, LANES)] = zvec
        return 0

    lax.fori_loop(0, CHUNK, zrow, 0)

    nzloc = (ZBLK - sid + NS - 1) // NS

    def zblk(i, _):
        off = pl.multiple_of((sid + i * NS) * CHUNK, 8)
        pltpu.sync_copy(rows_v[1], acc_sh.at[pl.ds(off, CHUNK)])
        return 0

    lax.fori_loop(0, nzloc, zblk, 0)

    @pl.when(sid == 0)
    def _():
        pltpu.sync_copy(rows_v[1].at[pl.ds(0, ZTAIL)],
                        acc_sh.at[pl.ds(ZBLK * CHUNK, ZTAIL)])

    pltpu.sync_copy(rall_hbm, rtab_v)

    wait_meta(0, 0)
    pltpu.async_copy(x_hbm.at[idx_v[0].at[0]], rows_v[0], semG[0])
    plsc.subcore_barrier()

    def quad_body(jj, _):
        for b in range(NBUF):
            bn = (b + 1) % NBUF
            i = NBUF * jj + b

            @pl.when(i < nloc)
            def _():
                pltpu.make_async_copy(
                    x_hbm.at[idx_v[b].at[0]], rows_v[b], semG[b]).wait()

                for d in range(CHUNK // LANES):
                    sl = pl.ds(d * LANES, LANES)
                    dst_v[b][sl] = idx_v[b][1, sl]

                @pl.when(i + 1 < nloc)
                def _():
                    wait_meta(i + 1, bn)

                    @pl.when(i + 1 >= NBUF)
                    def _():
                        pltpu.make_async_copy(
                            rows_v[bn], acc_sh.at[dst_v[bn]],
                            semS[bn]).wait()

                    pltpu.async_copy(
                        x_hbm.at[idx_v[bn].at[0]], rows_v[bn], semG[bn])

                @plsc.parallel_loop(0, CHUNK // LANES, unroll=4)
                def group_body(g):
                    gs = pl.ds(g * LANES, LANES)
                    tvec = idx_v[b][2, gs]
                    svec = scl_v[b][0, gs] * scl_v[b][1, gs]
                    base = g * LANES
                    for e in range(LANES):
                        t = tvec[e]
                        s = svec[e]
                        row = base + e
                        for d in range(D // LANES):
                            sl = pl.ds(d * LANES, LANES)
                            rows_v[b][row, sl] = (
                                rows_v[b][row, sl] * rtab_v[t, sl] * s)

                pltpu.async_copy(
                    rows_v[b], acc_sh.at[dst_v[b]], semS[b], add=True)

                @pl.when(i + NBUF < nloc)
                def _():
                    fire_meta(i + NBUF, b)

        return 0

    lax.fori_loop(0, (nloc + NBUF - 1) // NBUF, quad_body, 0)

    for b in range(NBUF):
        pltpu.make_async_copy(
            rows_v[b], acc_sh.at[dst_v[b]], semS[b]).wait()
    plsc.subcore_barrier()

    rsl = pl.ds(pl.multiple_of(sid * WBLK, 8), WBLK)
    tsl = pl.ds(NS * WBLK, WTAIL)

    @pl.when(cid == 0)
    def _():
        pltpu.sync_copy(acc_sh.at[rsl], out_in.at[rsl])

    @pl.when(jnp.logical_and(cid == 0, sid == 0))
    def _():
        pltpu.sync_copy(acc_sh.at[tsl], out_in.at[tsl])

    @pl.when(cid == 1)
    def _():
        pltpu.sync_copy(acc_sh.at[rsl], out_out.at[rsl])

    @pl.when(jnp.logical_and(cid == 1, sid == 0))
    def _():
        pltpu.sync_copy(acc_sh.at[tsl], out_out.at[tsl])


_ROWS_BLK = 1000


def _dense_body(a0_ref, a1_ref, x_ref, lr_ref, win_ref, wout_ref, wloop_ref,
                wrel_ref, bias_ref, rin_ref, xo_ref, ro_ref):
    acc = jnp.dot(a0_ref[...], win_ref[...], preferred_element_type=jnp.float32)
    acc += jnp.dot(a1_ref[...], wout_ref[...], preferred_element_type=jnp.float32)
    loop_in = x_ref[...] * lr_ref[...]
    acc += jnp.dot(loop_in, wloop_ref[...], preferred_element_type=jnp.float32)
    xo_ref[...] = jnp.tanh(acc * (1.0 / 3.0) + bias_ref[...])
    ro_ref[...] = jnp.dot(rin_ref[...], wrel_ref[...],
                          preferred_element_type=jnp.float32)


def _dense_layer(a0, a1, x, loop_rel, w_in, w_out, w_loop, w_rel, bias, r_in):
    nrel = r_in.shape[0]
    grid = (N // _ROWS_BLK,)
    row_spec = pl.BlockSpec((_ROWS_BLK, D), lambda i: (i, 0))
    full = pl.BlockSpec((D, D), lambda i: (0, 0))
    vec = pl.BlockSpec((1, D), lambda i: (0, 0))
    relspec = pl.BlockSpec((nrel, D), lambda i: (0, 0))
    return pl.pallas_call(
        _dense_body,
        grid=grid,
        in_specs=[row_spec, row_spec, row_spec, vec, full, full, full, full,
                  vec, relspec],
        out_specs=[row_spec, relspec],
        out_shape=[
            jax.ShapeDtypeStruct((N, D), jnp.float32),
            jax.ShapeDtypeStruct((nrel, D), jnp.float32),
        ],
    )(a0, a1, x, loop_rel, w_in, w_out, w_loop, w_rel,
      bias.reshape(1, D), r_in)


_B = 1024
_BPW = _B // (NC * NS)


@functools.partial(
    pl.kernel,
    out_type=(
        jax.ShapeDtypeStruct((_B, D), jnp.float32),
        jax.ShapeDtypeStruct((_B, D), jnp.float32),
    ),
    mesh=_sc_mesh(),
    scratch_types=[
        pltpu.VMEM((_BPW,), jnp.int32),
        pltpu.VMEM((_BPW,), jnp.int32),
        pltpu.VMEM((_BPW, D), jnp.float32),
        pltpu.VMEM((_BPW, D), jnp.float32),
        pltpu.SemaphoreType.DMA,
    ],
)
def _sc_take(x2_hbm, r2_hbm, subj_hbm, rel_hbm, sub_out, rel_out,
             si_v, ri_v, srow_v, rrow_v, sem):
    cid = lax.axis_index("c")
    sid = lax.axis_index("s")
    wid = sid * NC + cid
    base = wid * _BPW
    pltpu.sync_copy(subj_hbm.at[pl.ds(base, _BPW)], si_v)
    pltpu.async_copy(x2_hbm.at[si_v], srow_v, sem).wait()
    pltpu.sync_copy(srow_v, sub_out.at[pl.ds(base, _BPW)])
    pltpu.sync_copy(rel_hbm.at[pl.ds(base, _BPW)], ri_v)
    pltpu.async_copy(r2_hbm.at[ri_v], rrow_v, sem).wait()
    pltpu.sync_copy(rrow_v, rel_out.at[pl.ds(base, _BPW)])


def kernel(nf, edge_index, edge_type, edge_norm, edge_weight, subj, rel,
           init_rel, w_in1, w_out1, w_loop1, w_rel1, bias1, loop_rel1,
           w_in2, w_out2, w_loop2, w_rel2, bias2, loop_rel2):
    eidx = jnp.transpose(
        jnp.stack([edge_index[0], edge_index[1], edge_type])
        .reshape(3, NCHUNKS_TOT, CHUNK), (1, 0, 2))
    escale = jnp.transpose(
        jnp.stack([edge_norm, edge_weight])
        .reshape(2, NCHUNKS_TOT, CHUNK), (1, 0, 2))

    r_all1 = jnp.concatenate([init_rel, loop_rel1], axis=0)
    a0, a1 = _sc_aggregate(nf, r_all1, eidx, escale)
    x1, r1 = _dense_layer(a0, a1, nf, loop_rel1, w_in1, w_out1, w_loop1,
                          w_rel1, bias1, init_rel)

    r_all2 = jnp.concatenate([r1, loop_rel2], axis=0)
    b0, b1 = _sc_aggregate(x1, r_all2, eidx, escale)
    x2, r2 = _dense_layer(b0, b1, x1, loop_rel2, w_in2, w_out2, w_loop2,
                          w_rel2, bias2, r1)

    sub_emb, rel_emb = _sc_take(x2, r2, subj, rel)
    return sub_emb, rel_emb, x2


# speedup vs baseline: 1.3633x; 1.3633x over previous
import functools

import jax
import jax.numpy as jnp
from jax import lax
from jax.experimental import pallas as pl
from jax.experimental.pallas import tpu as pltpu
from jax.experimental.pallas import tpu_sc as plsc

N = 10000
E = 320000
D = 128
HALF = E // 2

NC = 2
NS = 16
LANES = 16

CHUNK = 128
NCHUNKS = HALF // CHUNK
ZROWS = 80
NZBLK = N // ZROWS
WBLK = 624
WTAIL = N - NS * WBLK


def _sc_mesh():
    return plsc.VectorSubcoreMesh(core_axis_name="c", subcore_axis_name="s")


NCHUNKS_TOT = E // CHUNK
NBUF = 2
ZBLK = N // CHUNK
ZTAIL = N - ZBLK * CHUNK


@functools.partial(
    pl.kernel,
    out_type=(
        jax.ShapeDtypeStruct((N, D), jnp.float32),
        jax.ShapeDtypeStruct((N, D), jnp.float32),
    ),
    mesh=_sc_mesh(),
    scratch_types=(
        [pltpu.VMEM((3, CHUNK), jnp.int32)] * NBUF
        + [pltpu.VMEM((2, CHUNK), jnp.float32)] * NBUF
        + [pltpu.VMEM((CHUNK, D), jnp.float32)] * NBUF
        + [pltpu.VMEM((CHUNK,), jnp.int32)] * NBUF
        + [
            pltpu.VMEM((41, D), jnp.float32),
            pltpu.VMEM_SHARED((N, D), jnp.float32),
        ]
        + [pltpu.SemaphoreType.DMA] * (3 * NBUF)
    ),
)
def _sc_aggregate(x_hbm, rall_hbm, eidx_hbm, escale_hbm,
                  out_in, out_out, *scratch):
    idx_v = scratch[0:NBUF]
    scl_v = scratch[NBUF:2 * NBUF]
    rows_v = scratch[2 * NBUF:3 * NBUF]
    dst_v = scratch[3 * NBUF:4 * NBUF]
    rtab_v, acc_sh = scratch[4 * NBUF:4 * NBUF + 2]
    semM = scratch[4 * NBUF + 2:4 * NBUF + 2 + NBUF]
    semG = scratch[4 * NBUF + 2 + NBUF:4 * NBUF + 2 + 2 * NBUF]
    semS = scratch[4 * NBUF + 2 + 2 * NBUF:4 * NBUF + 2 + 3 * NBUF]

    cid = lax.axis_index("c")
    sid = lax.axis_index("s")

    nloc = (NCHUNKS - sid + NS - 1) // NS

    def chunk_idx(i):
        return cid * NCHUNKS + sid + i * NS

    def fire_meta(i, b):
        c = chunk_idx(i)
        pltpu.async_copy(eidx_hbm.at[c], idx_v[b], semM[b])
        pltpu.async_copy(escale_hbm.at[c], scl_v[b], semM[b])

    def wait_meta(i, b):
        c = chunk_idx(i)
        pltpu.make_async_copy(eidx_hbm.at[c], idx_v[b], semM[b]).wait()
        pltpu.make_async_copy(escale_hbm.at[c], scl_v[b], semM[b]).wait()

    for b in range(NBUF):
        fire_meta(b, b)

    zvec = jnp.zeros((LANES,), jnp.float32)

    def zrow(i, _):
        for d in range(D // LANES):
            rows_v[1][i, pl.ds(d * LANES, LANES)] = zvec
        return 0

    lax.fori_loop(0, CHUNK, zrow, 0)

    nzloc = (ZBLK - sid + NS - 1) // NS

    def zblk(i, _):
        off = pl.multiple_of((sid + i * NS) * CHUNK, 8)
        pltpu.sync_copy(rows_v[1], acc_sh.at[pl.ds(off, CHUNK)])
        return 0

    lax.fori_loop(0, nzloc, zblk, 0)

    @pl.when(sid == 0)
    def _():
        pltpu.sync_copy(rows_v[1].at[pl.ds(0, ZTAIL)],
                        acc_sh.at[pl.ds(ZBLK * CHUNK, ZTAIL)])

    pltpu.sync_copy(rall_hbm, rtab_v)

    wait_meta(0, 0)
    pltpu.async_copy(x_hbm.at[idx_v[0].at[0]], rows_v[0], semG[0])
    plsc.subcore_barrier()

    def quad_body(jj, _):
        for b in range(NBUF):
            bn = (b + 1) % NBUF
            i = NBUF * jj + b

            @pl.when(i < nloc)
            def _():
                pltpu.make_async_copy(
                    x_hbm.at[idx_v[b].at[0]], rows_v[b], semG[b]).wait()

                for d in range(CHUNK // LANES):
                    sl = pl.ds(d * LANES, LANES)
                    dst_v[b][sl] = idx_v[b][1, sl]

                @pl.when(i + 1 < nloc)
                def _():
                    wait_meta(i + 1, bn)

                    @pl.when(i + 1 >= NBUF)
                    def _():
                        pltpu.make_async_copy(
                            rows_v[bn], acc_sh.at[dst_v[bn]],
                            semS[bn]).wait()

                    pltpu.async_copy(
                        x_hbm.at[idx_v[bn].at[0]], rows_v[bn], semG[bn])

                @plsc.parallel_loop(0, CHUNK // LANES, unroll=2)
                def group_body(g):
                    gs = pl.ds(g * LANES, LANES)
                    tvec = idx_v[b][2, gs]
                    svec = scl_v[b][0, gs] * scl_v[b][1, gs]
                    base = g * LANES
                    for e in range(LANES):
                        t = tvec[e]
                        s = svec[e]
                        row = base + e
                        for d in range(D // LANES):
                            sl = pl.ds(d * LANES, LANES)
                            rows_v[b][row, sl] = (
                                rows_v[b][row, sl] * rtab_v[t, sl] * s)

                pltpu.async_copy(
                    rows_v[b], acc_sh.at[dst_v[b]], semS[b], add=True)

                @pl.when(i + NBUF < nloc)
                def _():
                    fire_meta(i + NBUF, b)

        return 0

    lax.fori_loop(0, (nloc + NBUF - 1) // NBUF, quad_body, 0)

    for b in range(NBUF):
        pltpu.make_async_copy(
            rows_v[b], acc_sh.at[dst_v[b]], semS[b]).wait()
    plsc.subcore_barrier()

    rsl = pl.ds(pl.multiple_of(sid * WBLK, 8), WBLK)
    tsl = pl.ds(NS * WBLK, WTAIL)

    @pl.when(cid == 0)
    def _():
        pltpu.sync_copy(acc_sh.at[rsl], out_in.at[rsl])

    @pl.when(jnp.logical_and(cid == 0, sid == 0))
    def _():
        pltpu.sync_copy(acc_sh.at[tsl], out_in.at[tsl])

    @pl.when(cid == 1)
    def _():
        pltpu.sync_copy(acc_sh.at[rsl], out_out.at[rsl])

    @pl.when(jnp.logical_and(cid == 1, sid == 0))
    def _():
        pltpu.sync_copy(acc_sh.at[tsl], out_out.at[tsl])


_ROWS_BLK = 1000


def _dense_body(a0_ref, a1_ref, x_ref, lr_ref, win_ref, wout_ref, wloop_ref,
                wrel_ref, bias_ref, rin_ref, xo_ref, ro_ref):
    acc = jnp.dot(a0_ref[...], win_ref[...], preferred_element_type=jnp.float32)
    acc += jnp.dot(a1_ref[...], wout_ref[...], preferred_element_type=jnp.float32)
    loop_in = x_ref[...] * lr_ref[...]
    acc += jnp.dot(loop_in, wloop_ref[...], preferred_element_type=jnp.float32)
    xo_ref[...] = jnp.tanh(acc * (1.0 / 3.0) + bias_ref[...])
    ro_ref[...] = jnp.dot(rin_ref[...], wrel_ref[...],
                          preferred_element_type=jnp.float32)


def _dense_layer(a0, a1, x, loop_rel, w_in, w_out, w_loop, w_rel, bias, r_in):
    nrel = r_in.shape[0]
    grid = (N // _ROWS_BLK,)
    row_spec = pl.BlockSpec((_ROWS_BLK, D), lambda i: (i, 0))
    full = pl.BlockSpec((D, D), lambda i: (0, 0))
    vec = pl.BlockSpec((1, D), lambda i: (0, 0))
    relspec = pl.BlockSpec((nrel, D), lambda i: (0, 0))
    return pl.pallas_call(
        _dense_body,
        grid=grid,
        in_specs=[row_spec, row_spec, row_spec, vec, full, full, full, full,
                  vec, relspec],
        out_specs=[row_spec, relspec],
        out_shape=[
            jax.ShapeDtypeStruct((N, D), jnp.float32),
            jax.ShapeDtypeStruct((nrel, D), jnp.float32),
        ],
    )(a0, a1, x, loop_rel, w_in, w_out, w_loop, w_rel,
      bias.reshape(1, D), r_in)


_B = 1024
_BPW = _B // (NC * NS)


@functools.partial(
    pl.kernel,
    out_type=(
        jax.ShapeDtypeStruct((_B, D), jnp.float32),
        jax.ShapeDtypeStruct((_B, D), jnp.float32),
    ),
    mesh=_sc_mesh(),
    scratch_types=[
        pltpu.VMEM((_BPW,), jnp.int32),
        pltpu.VMEM((_BPW,), jnp.int32),
        pltpu.VMEM((_BPW, D), jnp.float32),
        pltpu.VMEM((_BPW, D), jnp.float32),
        pltpu.SemaphoreType.DMA,
    ],
)
def _sc_take(x2_hbm, r2_hbm, subj_hbm, rel_hbm, sub_out, rel_out,
             si_v, ri_v, srow_v, rrow_v, sem):
    cid = lax.axis_index("c")
    sid = lax.axis_index("s")
    wid = sid * NC + cid
    base = wid * _BPW
    pltpu.sync_copy(subj_hbm.at[pl.ds(base, _BPW)], si_v)
    pltpu.async_copy(x2_hbm.at[si_v], srow_v, sem).wait()
    pltpu.sync_copy(srow_v, sub_out.at[pl.ds(base, _BPW)])
    pltpu.sync_copy(rel_hbm.at[pl.ds(base, _BPW)], ri_v)
    pltpu.async_copy(r2_hbm.at[ri_v], rrow_v, sem).wait()
    pltpu.sync_copy(rrow_v, rel_out.at[pl.ds(base, _BPW)])


def kernel(nf, edge_index, edge_type, edge_norm, edge_weight, subj, rel,
           init_rel, w_in1, w_out1, w_loop1, w_rel1, bias1, loop_rel1,
           w_in2, w_out2, w_loop2, w_rel2, bias2, loop_rel2):
    eidx = jnp.transpose(
        jnp.stack([edge_index[0], edge_index[1], edge_type])
        .reshape(3, NCHUNKS_TOT, CHUNK), (1, 0, 2))
    escale = jnp.transpose(
        jnp.stack([edge_norm, edge_weight])
        .reshape(2, NCHUNKS_TOT, CHUNK), (1, 0, 2))

    r_all1 = jnp.concatenate([init_rel, loop_rel1], axis=0)
    a0, a1 = _sc_aggregate(nf, r_all1, eidx, escale)
    x1, r1 = _dense_layer(a0, a1, nf, loop_rel1, w_in1, w_out1, w_loop1,
                          w_rel1, bias1, init_rel)

    r_all2 = jnp.concatenate([r1, loop_rel2], axis=0)
    b0, b1 = _sc_aggregate(x1, r_all2, eidx, escale)
    x2, r2 = _dense_layer(b0, b1, x1, loop_rel2, w_in2, w_out2, w_loop2,
                          w_rel2, bias2, r1)

    sub_emb, rel_emb = _sc_take(x2, r2, subj, rel)
    return sub_emb, rel_emb, x2


# speedup vs baseline: 1.3633x; 1.0000x over previous
import functools

import jax
import jax.numpy as jnp
from jax import lax
from jax.experimental import pallas as pl
from jax.experimental.pallas import tpu as pltpu
from jax.experimental.pallas import tpu_sc as plsc

N = 10000
E = 320000
D = 128
HALF = E // 2

NC = 2
NS = 16
LANES = 16

CHUNK = 128
NCHUNKS = HALF // CHUNK
ZROWS = 80
NZBLK = N // ZROWS
WBLK = 624
WTAIL = N - NS * WBLK


def _sc_mesh():
    return plsc.VectorSubcoreMesh(core_axis_name="c", subcore_axis_name="s")


NCHUNKS_TOT = E // CHUNK
NBUF = 2
ZBLK = N // CHUNK
ZTAIL = N - ZBLK * CHUNK


@functools.partial(
    pl.kernel,
    out_type=(
        jax.ShapeDtypeStruct((N, D), jnp.float32),
        jax.ShapeDtypeStruct((N, D), jnp.float32),
    ),
    mesh=_sc_mesh(),
    scratch_types=(
        [pltpu.VMEM((3, CHUNK), jnp.int32)] * NBUF
        + [pltpu.VMEM((2, CHUNK), jnp.float32)] * NBUF
        + [pltpu.VMEM((CHUNK, D), jnp.float32)] * NBUF
        + [pltpu.VMEM((CHUNK,), jnp.int32)] * NBUF
        + [
            pltpu.VMEM((41, D), jnp.float32),
            pltpu.VMEM_SHARED((N, D), jnp.float32),
        ]
        + [pltpu.SemaphoreType.DMA] * (3 * NBUF)
    ),
)
def _sc_aggregate(x_hbm, rall_hbm, eidx_hbm, escale_hbm,
                  out_in, out_out, *scratch):
    idx_v = scratch[0:NBUF]
    scl_v = scratch[NBUF:2 * NBUF]
    rows_v = scratch[2 * NBUF:3 * NBUF]
    dst_v = scratch[3 * NBUF:4 * NBUF]
    rtab_v, acc_sh = scratch[4 * NBUF:4 * NBUF + 2]
    semM = scratch[4 * NBUF + 2:4 * NBUF + 2 + NBUF]
    semG = scratch[4 * NBUF + 2 + NBUF:4 * NBUF + 2 + 2 * NBUF]
    semS = scratch[4 * NBUF + 2 + 2 * NBUF:4 * NBUF + 2 + 3 * NBUF]

    cid = lax.axis_index("c")
    sid = lax.axis_index("s")

    nloc = (NCHUNKS - sid + NS - 1) // NS

    def chunk_idx(i):
        return cid * NCHUNKS + sid + i * NS

    def fire_meta(i, b):
        c = chunk_idx(i)
        pltpu.async_copy(eidx_hbm.at[c], idx_v[b], semM[b])
        pltpu.async_copy(escale_hbm.at[c], scl_v[b], semM[b])

    def wait_meta(i, b):
        c = chunk_idx(i)
        pltpu.make_async_copy(eidx_hbm.at[c], idx_v[b], semM[b]).wait()
        pltpu.make_async_copy(escale_hbm.at[c], scl_v[b], semM[b]).wait()

    for b in range(NBUF):
        fire_meta(b, b)

    zvec = jnp.zeros((LANES,), jnp.float32)

    def zrow(i, _):
        for d in range(D // LANES):
            rows_v[1][i, pl.ds(d * LANES, LANES)] = zvec
        return 0

    lax.fori_loop(0, CHUNK, zrow, 0)

    nzloc = (ZBLK - sid + NS - 1) // NS

    def zblk(i, _):
        off = pl.multiple_of((sid + i * NS) * CHUNK, 8)
        pltpu.sync_copy(rows_v[1], acc_sh.at[pl.ds(off, CHUNK)])
        return 0

    lax.fori_loop(0, nzloc, zblk, 0)

    @pl.when(sid == 0)
    def _():
        pltpu.sync_copy(rows_v[1].at[pl.ds(0, ZTAIL)],
                        acc_sh.at[pl.ds(ZBLK * CHUNK, ZTAIL)])

    pltpu.sync_copy(rall_hbm, rtab_v)

    wait_meta(0, 0)
    pltpu.async_copy(x_hbm.at[idx_v[0].at[0]], rows_v[0], semG[0])
    plsc.subcore_barrier()

    def quad_body(jj, _):
        for b in range(NBUF):
            bn = (b + 1) % NBUF
            i = NBUF * jj + b

            @pl.when(i < nloc)
            def _():
                pltpu.make_async_copy(
                    x_hbm.at[idx_v[b].at[0]], rows_v[b], semG[b]).wait()

                for d in range(CHUNK // LANES):
                    sl = pl.ds(d * LANES, LANES)
                    dst_v[b][sl] = idx_v[b][1, sl]

                @pl.when(i + 1 < nloc)
                def _():
                    wait_meta(i + 1, bn)

                    @pl.when(i + 1 >= NBUF)
                    def _():
                        pltpu.make_async_copy(
                            rows_v[bn], acc_sh.at[dst_v[bn]],
                            semS[bn]).wait()

                    pltpu.async_copy(
                        x_hbm.at[idx_v[bn].at[0]], rows_v[bn], semG[bn])

                @plsc.parallel_loop(0, CHUNK // LANES, unroll=2)
                def group_body(g):
                    gs = pl.ds(g * LANES, LANES)
                    tvec = idx_v[b][2, gs]
                    svec = scl_v[b][0, gs] * scl_v[b][1, gs]
                    base = g * LANES
                    for e in range(LANES):
                        t = tvec[e]
                        s_vec = jnp.full((LANES,), svec[e], jnp.float32)
                        row = base + e
                        for d in range(D // LANES):
                            sl = pl.ds(d * LANES, LANES)
                            rows_v[b][row, sl] = (
                                rows_v[b][row, sl] * rtab_v[t, sl] * s_vec)

                pltpu.async_copy(
                    rows_v[b], acc_sh.at[dst_v[b]], semS[b], add=True)

                @pl.when(i + NBUF < nloc)
                def _():
                    fire_meta(i + NBUF, b)

        return 0

    lax.fori_loop(0, (nloc + NBUF - 1) // NBUF, quad_body, 0)

    for b in range(NBUF):
        pltpu.make_async_copy(
            rows_v[b], acc_sh.at[dst_v[b]], semS[b]).wait()
    plsc.subcore_barrier()

    rsl = pl.ds(pl.multiple_of(sid * WBLK, 8), WBLK)
    tsl = pl.ds(NS * WBLK, WTAIL)

    @pl.when(cid == 0)
    def _():
        pltpu.sync_copy(acc_sh.at[rsl], out_in.at[rsl])

    @pl.when(jnp.logical_and(cid == 0, sid == 0))
    def _():
        pltpu.sync_copy(acc_sh.at[tsl], out_in.at[tsl])

    @pl.when(cid == 1)
    def _():
        pltpu.sync_copy(acc_sh.at[rsl], out_out.at[rsl])

    @pl.when(jnp.logical_and(cid == 1, sid == 0))
    def _():
        pltpu.sync_copy(acc_sh.at[tsl], out_out.at[tsl])


_ROWS_BLK = 1000


def _dense_body(a0_ref, a1_ref, x_ref, lr_ref, win_ref, wout_ref, wloop_ref,
                wrel_ref, bias_ref, rin_ref, xo_ref, ro_ref):
    acc = jnp.dot(a0_ref[...], win_ref[...], preferred_element_type=jnp.float32)
    acc += jnp.dot(a1_ref[...], wout_ref[...], preferred_element_type=jnp.float32)
    loop_in = x_ref[...] * lr_ref[...]
    acc += jnp.dot(loop_in, wloop_ref[...], preferred_element_type=jnp.float32)
    xo_ref[...] = jnp.tanh(acc * (1.0 / 3.0) + bias_ref[...])
    ro_ref[...] = jnp.dot(rin_ref[...], wrel_ref[...],
                          preferred_element_type=jnp.float32)


def _dense_layer(a0, a1, x, loop_rel, w_in, w_out, w_loop, w_rel, bias, r_in):
    nrel = r_in.shape[0]
    grid = (N // _ROWS_BLK,)
    row_spec = pl.BlockSpec((_ROWS_BLK, D), lambda i: (i, 0))
    full = pl.BlockSpec((D, D), lambda i: (0, 0))
    vec = pl.BlockSpec((1, D), lambda i: (0, 0))
    relspec = pl.BlockSpec((nrel, D), lambda i: (0, 0))
    return pl.pallas_call(
        _dense_body,
        grid=grid,
        in_specs=[row_spec, row_spec, row_spec, vec, full, full, full, full,
                  vec, relspec],
        out_specs=[row_spec, relspec],
        out_shape=[
            jax.ShapeDtypeStruct((N, D), jnp.float32),
            jax.ShapeDtypeStruct((nrel, D), jnp.float32),
        ],
    )(a0, a1, x, loop_rel, w_in, w_out, w_loop, w_rel,
      bias.reshape(1, D), r_in)


_B = 1024
_BPW = _B // (NC * NS)


@functools.partial(
    pl.kernel,
    out_type=(
        jax.ShapeDtypeStruct((_B, D), jnp.float32),
        jax.ShapeDtypeStruct((_B, D), jnp.float32),
    ),
    mesh=_sc_mesh(),
    scratch_types=[
        pltpu.VMEM((_BPW,), jnp.int32),
        pltpu.VMEM((_BPW,), jnp.int32),
        pltpu.VMEM((_BPW, D), jnp.float32),
        pltpu.VMEM((_BPW, D), jnp.float32),
        pltpu.SemaphoreType.DMA,
    ],
)
def _sc_take(x2_hbm, r2_hbm, subj_hbm, rel_hbm, sub_out, rel_out,
             si_v, ri_v, srow_v, rrow_v, sem):
    cid = lax.axis_index("c")
    sid = lax.axis_index("s")
    wid = sid * NC + cid
    base = wid * _BPW
    pltpu.sync_copy(subj_hbm.at[pl.ds(base, _BPW)], si_v)
    pltpu.async_copy(x2_hbm.at[si_v], srow_v, sem).wait()
    pltpu.sync_copy(srow_v, sub_out.at[pl.ds(base, _BPW)])
    pltpu.sync_copy(rel_hbm.at[pl.ds(base, _BPW)], ri_v)
    pltpu.async_copy(r2_hbm.at[ri_v], rrow_v, sem).wait()
    pltpu.sync_copy(rrow_v, rel_out.at[pl.ds(base, _BPW)])


def kernel(nf, edge_index, edge_type, edge_norm, edge_weight, subj, rel,
           init_rel, w_in1, w_out1, w_loop1, w_rel1, bias1, loop_rel1,
           w_in2, w_out2, w_loop2, w_rel2, bias2, loop_rel2):
    eidx = jnp.transpose(
        jnp.stack([edge_index[0], edge_index[1], edge_type])
        .reshape(3, NCHUNKS_TOT, CHUNK), (1, 0, 2))
    escale = jnp.transpose(
        jnp.stack([edge_norm, edge_weight])
        .reshape(2, NCHUNKS_TOT, CHUNK), (1, 0, 2))

    r_all1 = jnp.concatenate([init_rel, loop_rel1], axis=0)
    a0, a1 = _sc_aggregate(nf, r_all1, eidx, escale)
    x1, r1 = _dense_layer(a0, a1, nf, loop_rel1, w_in1, w_out1, w_loop1,
                          w_rel1, bias1, init_rel)

    r_all2 = jnp.concatenate([r1, loop_rel2], axis=0)
    b0, b1 = _sc_aggregate(x1, r_all2, eidx, escale)
    x2, r2 = _dense_layer(b0, b1, x1, loop_rel2, w_in2, w_out2, w_loop2,
                          w_rel2, bias2, r1)

    sub_emb, rel_emb = _sc_take(x2, r2, subj, rel)
    return sub_emb, rel_emb, x2
